# baseline (speedup 1.0000x reference)
import functools

import jax
import jax.numpy as jnp
from jax import lax
from jax.experimental import pallas as pl
from jax.experimental.pallas import tpu as pltpu
from jax.experimental.pallas import tpu_sc as plsc

D = 32
NC = 2
NS = 16
NW = NC * NS
CHUNK = 128


def _sc_gathern(B, plan_ids):
    b_per_w = B // NW
    n_chunks = b_per_w // CHUNK
    n_out = len(plan_ids)
    mesh = plsc.VectorSubcoreMesh(core_axis_name="c", subcore_axis_name="s")
    out_type = [jax.ShapeDtypeStruct((B, D), jnp.bfloat16)] * n_out
    scratch_types = [
        pltpu.VMEM((n_chunks, CHUNK), jnp.int32),
        pltpu.VMEM((n_chunks, CHUNK), jnp.int32),
        pltpu.VMEM((n_chunks, CHUNK), jnp.int32),
        pltpu.VMEM((b_per_w, D), jnp.bfloat16),
        pltpu.VMEM((b_per_w, D), jnp.bfloat16),
        pltpu.SemaphoreType.DMA,
        pltpu.SemaphoreType.DMA,
    ]

    @functools.partial(pl.kernel, mesh=mesh, out_type=out_type,
                       scratch_types=scratch_types,
                       compiler_params=pltpu.CompilerParams(
                           use_tc_tiling_on_sc=False))
    def k(*refs):
        tbls = refs[0:3]
        ih = refs[3:6]
        outs = refs[6:6 + n_out]
        iv = refs[6 + n_out:9 + n_out]
        buf0, buf1, sem0, sem1 = refs[9 + n_out:13 + n_out]
        wid = lax.axis_index("s") * NC + lax.axis_index("c")
        base = wid * b_per_w
        for h, v in zip(ih, iv):
            pltpu.sync_copy(h.at[wid], v)

        plan = [(tbls[ti], iv[ii]) for ti, ii in plan_ids]
        bufs = (buf0, buf1)
        sems = (sem0, sem1)

        def fire(g):
            tbl, idx = plan[g]
            buf, sem = bufs[g % 2], sems[g % 2]
            return [pltpu.async_copy(tbl.at[idx.at[c]],
                                     buf.at[pl.ds(c * CHUNK, CHUNK)], sem)
                    for c in range(n_chunks)]

        pending = fire(0)
        for g in range(n_out):
            for h in pending:
                h.wait()
            if g < n_out - 1:
                nxt = fire(g + 1)
            pltpu.sync_copy(bufs[g % 2], outs[g].at[pl.ds(base, b_per_w)])
            if g < n_out - 1:
                pending = nxt

    return k


def _tc_combine(nb):
    def body(au, tu, ai, ti, waau, watu, wbau, wbtu, ar, tr,
             w1a, b1a, w2a, b2a, w1t, b1t, w2t, b2t,
             w1i, b1i, w2i, b2i, w1j, b1j, w2j, b2j, la, lt):
        f32 = jnp.float32
        v_au, v_tu = au[...].astype(f32), tu[...].astype(f32)
        g_waau, g_watu = waau[...].astype(f32), watu[...].astype(f32)
        g_wbau, g_wbtu = wbau[...].astype(f32), wbtu[...].astype(f32)
        x_au = g_waau * v_au + (1.0 - g_watu) * v_tu
        x_tu = g_wbau * v_au + (1.0 - g_wbtu) * v_tu

        def mlp(x, w1, b1, w2, b2):
            h = jnp.maximum(
                jnp.dot(x, w1[...], preferred_element_type=jnp.float32)
                + b1[...], 0.0)
            return jnp.maximum(
                jnp.dot(h, w2[...], preferred_element_type=jnp.float32)
                + b2[...], 0.0)

        f_au = mlp(x_au, w1a, b1a, w2a, b2a)
        f_tu = mlp(x_tu, w1t, b1t, w2t, b2t)
        f_ai = mlp(ai[...].astype(f32), w1i, b1i, w2i, b2i)
        f_ti = mlp(ti[...].astype(f32), w1j, b1j, w2j, b2j)

        a_dot = jnp.sum(f_au * f_ai, axis=1, keepdims=True)
        t_dot = jnp.sum(f_tu * f_ti, axis=1, keepdims=True)
        a_s = jnp.maximum(a_dot, jnp.float32(1e-06))
        t_s = jnp.maximum(t_dot, jnp.float32(1e-06))
        pa = jnp.sum((a_s - ar[...]) ** 2)
        pt = jnp.sum((t_s - tr[...]) ** 2)

        i = pl.program_id(0)

        @pl.when(i == 0)
        def _():
            la[0, 0] = jnp.float32(0.0)
            lt[0, 0] = jnp.float32(0.0)

        la[0, 0] += pa
        lt[0, 0] += pt

    return body


def kernel(ausers, aitems, aratings, tusers, titems, tratings, params):
    B = ausers.shape[0]
    assert B % (NW * CHUNK) == 0
    n_chunks = (B // NW) // CHUNK

    au3, tu3, ai3, ti3 = (a.astype(jnp.int32).reshape(NW, n_chunks, CHUNK)
                          for a in (ausers, tusers, aitems, titems))

    bf = jnp.bfloat16
    a_u, wa_au, wa_tu, wb_au, wb_tu = _sc_gathern(
        B, ((0, 0), (1, 0), (1, 1), (2, 0), (2, 1)))(
        params["a_emb_user"].astype(bf), params["W_a"].astype(bf),
        params["W_b"].astype(bf), au3, tu3, tu3)
    t_u, a_i, t_i = _sc_gathern(
        B, ((0, 0), (1, 1), (2, 2)))(
        params["t_emb_user"].astype(bf), params["a_emb_item"].astype(bf),
        params["t_emb_item"].astype(bf), tu3, ai3, ti3)

    NB = 8
    R = B // NB
    row = pl.BlockSpec((R, D), lambda i: (i, 0))
    col = pl.BlockSpec((R, 1), lambda i: (i, 0))

    def full(shape):
        return pl.BlockSpec(shape, lambda i: tuple(0 for _ in shape))

    mlps = (params["mlp_a_users"], params["mlp_t_users"],
            params["mlp_a_items"], params["mlp_t_items"])
    wargs, wspecs = [], []
    for p in mlps:
        for nm, arr in (("W1", p["W1"]), ("b1", p["b1"].reshape(1, -1)),
                        ("W2", p["W2"]), ("b2", p["b2"].reshape(1, -1))):
            wargs.append(arr)
            wspecs.append(full(arr.shape))

    ar2 = aratings.astype(jnp.float32).reshape(B, 1)
    tr2 = tratings.astype(jnp.float32).reshape(B, 1)

    la, lt = pl.pallas_call(
        _tc_combine(NB),
        grid=(NB,),
        in_specs=[row] * 8 + [col, col] + wspecs,
        out_specs=(pl.BlockSpec((1, 1), lambda i: (0, 0),
                                memory_space=pltpu.SMEM),) * 2,
        out_shape=(jax.ShapeDtypeStruct((1, 1), jnp.float32),) * 2,
    )(a_u, t_u, a_i, t_i, wa_au, wa_tu, wb_au, wb_tu, ar2, tr2, *wargs)

    inv_b = jnp.float32(1.0 / B)
    return (la[0, 0] * inv_b, lt[0, 0] * inv_b)


# speedup vs baseline: 1.3499x; 1.3499x over previous
import functools

import jax
import jax.numpy as jnp
from jax import lax
from jax.experimental import pallas as pl
from jax.experimental.pallas import tpu as pltpu
from jax.experimental.pallas import tpu_sc as plsc

D = 32
NC = 2
NS = 16
NW = NC * NS
CHUNK = 128


def _sc_gathern(B, plan_ids):
    b_per_w = B // NW
    n_chunks = b_per_w // CHUNK
    n_out = len(plan_ids)
    mesh = plsc.VectorSubcoreMesh(core_axis_name="c", subcore_axis_name="s")
    out_type = [jax.ShapeDtypeStruct((B, D), jnp.float32)] * n_out
    scratch_types = [
        pltpu.VMEM((n_chunks, CHUNK), jnp.int32),
        pltpu.VMEM((n_chunks, CHUNK), jnp.int32),
        pltpu.VMEM((n_chunks, CHUNK), jnp.int32),
        pltpu.VMEM((b_per_w, D), jnp.float32),
        pltpu.VMEM((b_per_w, D), jnp.float32),
        pltpu.SemaphoreType.DMA,
        pltpu.SemaphoreType.DMA,
    ]

    @functools.partial(pl.kernel, mesh=mesh, out_type=out_type,
                       scratch_types=scratch_types,
                       compiler_params=pltpu.CompilerParams(
                           use_tc_tiling_on_sc=False))
    def k(*refs):
        tbls = refs[0:3]
        ih = refs[3:6]
        outs = refs[6:6 + n_out]
        iv = refs[6 + n_out:9 + n_out]
        buf0, buf1, sem0, sem1 = refs[9 + n_out:13 + n_out]
        wid = lax.axis_index("s") * NC + lax.axis_index("c")
        base = wid * b_per_w
        for h, v in zip(ih, iv):
            pltpu.sync_copy(h.at[wid], v)

        plan = [(tbls[ti], iv[ii]) for ti, ii in plan_ids]
        bufs = (buf0, buf1)
        sems = (sem0, sem1)

        def fire(g):
            tbl, idx = plan[g]
            buf, sem = bufs[g % 2], sems[g % 2]
            return [pltpu.async_copy(tbl.at[idx.at[c]],
                                     buf.at[pl.ds(c * CHUNK, CHUNK)], sem)
                    for c in range(n_chunks)]

        pending = fire(0)
        for g in range(n_out):
            for h in pending:
                h.wait()
            if g < n_out - 1:
                nxt = fire(g + 1)
            pltpu.sync_copy(bufs[g % 2], outs[g].at[pl.ds(base, b_per_w)])
            if g < n_out - 1:
                pending = nxt

    return k


def _tc_combine(nb):
    def body(au, tu, ai, ti, waau, watu, wbau, wbtu, ar, tr,
             w1a, b1a, w2a, b2a, w1t, b1t, w2t, b2t,
             w1i, b1i, w2i, b2i, w1j, b1j, w2j, b2j, la, lt):
        x_au = waau[...] * au[...] + (1.0 - watu[...]) * tu[...]
        x_tu = wbau[...] * au[...] + (1.0 - wbtu[...]) * tu[...]

        def mlp(x, w1, b1, w2, b2):
            h = jnp.maximum(
                jnp.dot(x, w1[...], preferred_element_type=jnp.float32)
                + b1[...], 0.0)
            return jnp.maximum(
                jnp.dot(h, w2[...], preferred_element_type=jnp.float32)
                + b2[...], 0.0)

        f_au = mlp(x_au, w1a, b1a, w2a, b2a)
        f_tu = mlp(x_tu, w1t, b1t, w2t, b2t)
        f_ai = mlp(ai[...], w1i, b1i, w2i, b2i)
        f_ti = mlp(ti[...], w1j, b1j, w2j, b2j)

        a_dot = jnp.sum(f_au * f_ai, axis=1, keepdims=True)
        t_dot = jnp.sum(f_tu * f_ti, axis=1, keepdims=True)
        a_s = jnp.maximum(a_dot, jnp.float32(1e-06))
        t_s = jnp.maximum(t_dot, jnp.float32(1e-06))
        pa = jnp.sum((a_s - ar[...]) ** 2)
        pt = jnp.sum((t_s - tr[...]) ** 2)

        i = pl.program_id(0)

        @pl.when(i == 0)
        def _():
            la[0, 0] = jnp.float32(0.0)
            lt[0, 0] = jnp.float32(0.0)

        la[0, 0] += pa
        lt[0, 0] += pt

    return body


def kernel(ausers, aitems, aratings, tusers, titems, tratings, params):
    B = ausers.shape[0]
    assert B % (NW * CHUNK) == 0
    n_chunks = (B // NW) // CHUNK

    au3, tu3, ai3, ti3 = (a.astype(jnp.int32).reshape(NW, n_chunks, CHUNK)
                          for a in (ausers, tusers, aitems, titems))

    a_u, wa_au, wa_tu, wb_au, wb_tu = _sc_gathern(
        B, ((0, 0), (1, 0), (1, 1), (2, 0), (2, 1)))(
        params["a_emb_user"], params["W_a"], params["W_b"],
        au3, tu3, tu3)
    t_u, a_i, t_i = _sc_gathern(
        B, ((0, 0), (1, 1), (2, 2)))(
        params["t_emb_user"], params["a_emb_item"], params["t_emb_item"],
        tu3, ai3, ti3)

    NB = 8
    R = B // NB
    row = pl.BlockSpec((R, D), lambda i: (i, 0))
    col = pl.BlockSpec((R, 1), lambda i: (i, 0))

    def full(shape):
        return pl.BlockSpec(shape, lambda i: tuple(0 for _ in shape))

    mlps = (params["mlp_a_users"], params["mlp_t_users"],
            params["mlp_a_items"], params["mlp_t_items"])
    wargs, wspecs = [], []
    for p in mlps:
        for nm, arr in (("W1", p["W1"]), ("b1", p["b1"].reshape(1, -1)),
                        ("W2", p["W2"]), ("b2", p["b2"].reshape(1, -1))):
            wargs.append(arr)
            wspecs.append(full(arr.shape))

    ar2 = aratings.astype(jnp.float32).reshape(B, 1)
    tr2 = tratings.astype(jnp.float32).reshape(B, 1)

    la, lt = pl.pallas_call(
        _tc_combine(NB),
        grid=(NB,),
        in_specs=[row] * 8 + [col, col] + wspecs,
        out_specs=(pl.BlockSpec((1, 1), lambda i: (0, 0),
                                memory_space=pltpu.SMEM),) * 2,
        out_shape=(jax.ShapeDtypeStruct((1, 1), jnp.float32),) * 2,
    )(a_u, t_u, a_i, t_i, wa_au, wa_tu, wb_au, wb_tu, ar2, tr2, *wargs)

    inv_b = jnp.float32(1.0 / B)
    return (la[0, 0] * inv_b, lt[0, 0] * inv_b)
